# Initial kernel scaffold; baseline (speedup 1.0000x reference)
import jax
import jax.numpy as jnp
from jax.experimental import pallas as pl


def kernel(input, adj, weight, weight2, gamma, beta):
    raise NotImplementedError("write your pallas kernel here")


# speedup vs baseline: 1.2869x; 1.2869x over previous
import jax
import jax.numpy as jnp
from jax.experimental import pallas as pl
from jax.experimental.pallas import tpu as pltpu

_N = 10000
_D = 512
_BM = 400


def _support_body(inp_ref, w_ref, out_ref):
    out_ref[...] = jnp.dot(
        inp_ref[...].astype(jnp.bfloat16),
        w_ref[...],
        preferred_element_type=jnp.float32,
    ).astype(jnp.bfloat16)


def _main_body(adj_ref, sup_ref, w2_ref, gamma_ref, beta_ref, out_ref):
    a = adj_ref[...].astype(jnp.bfloat16)
    h = jnp.dot(a, sup_ref[...], preferred_element_type=jnp.float32)
    h = jnp.maximum(h, 0.0).astype(jnp.bfloat16)
    o = jnp.dot(h, w2_ref[...], preferred_element_type=jnp.float32)
    mean = jnp.mean(o, axis=-1, keepdims=True)
    var = jnp.mean(jnp.square(o - mean), axis=-1, keepdims=True)
    o = (o - mean) * jax.lax.rsqrt(var + 1e-5) * gamma_ref[...] + beta_ref[...]
    out_ref[...] = jnp.maximum(o, 0.0)


def kernel(input, adj, weight, weight2, gamma, beta):
    w_bf16 = weight.astype(jnp.bfloat16)
    w2_bf16 = weight2.astype(jnp.bfloat16)
    gamma2d = gamma.reshape(1, _D)
    beta2d = beta.reshape(1, _D)

    support = pl.pallas_call(
        _support_body,
        grid=(5,),
        in_specs=[
            pl.BlockSpec((_N // 5, _D), lambda i: (i, 0)),
            pl.BlockSpec((_D, _D), lambda i: (0, 0)),
        ],
        out_specs=pl.BlockSpec((_N // 5, _D), lambda i: (i, 0)),
        out_shape=jax.ShapeDtypeStruct((_N, _D), jnp.bfloat16),
        compiler_params=pltpu.CompilerParams(
            dimension_semantics=("parallel",),
        ),
    )(input, w_bf16)

    out = pl.pallas_call(
        _main_body,
        grid=(_N // _BM,),
        in_specs=[
            pl.BlockSpec((_BM, _N), lambda i: (i, 0)),
            pl.BlockSpec((_N, _D), lambda i: (0, 0)),
            pl.BlockSpec((_D, _D), lambda i: (0, 0)),
            pl.BlockSpec((1, _D), lambda i: (0, 0)),
            pl.BlockSpec((1, _D), lambda i: (0, 0)),
        ],
        out_specs=pl.BlockSpec((_BM, _D), lambda i: (i, 0)),
        out_shape=jax.ShapeDtypeStruct((_N, _D), jnp.float32),
        compiler_params=pltpu.CompilerParams(
            dimension_semantics=("parallel",),
        ),
    )(adj, support, w2_bf16, gamma2d, beta2d)
    return out
